# baseline (speedup 1.0000x reference)
import functools

import numpy as np
import jax
import jax.numpy as jnp
from jax import lax
from jax.experimental import pallas as pl
from jax.experimental.pallas import tpu as pltpu
from jax.experimental.pallas import tpu_sc as plsc

_NUM_RBF = 128
_CUTOFF = 5.0
_DELTA = _CUTOFF / (_NUM_RBF - 1)
_COEFF = -0.5 / (2.0 * _DELTA) ** 2
_S2 = float(1.0 / np.sqrt(2.0))
_S6 = float(1.0 / np.sqrt(6.0))

_B2_NZ = (
    (((0, 1), _S2), ((1, 0), _S2)),
    (((1, 2), _S2), ((2, 1), _S2)),
    (((0, 0), -_S6), ((1, 1), -_S6), ((2, 2), 2.0 * _S6)),
    (((0, 2), _S2), ((2, 0), _S2)),
    (((0, 0), _S2), ((1, 1), -_S2)),
)


def _silu(x):
    return x * jax.nn.sigmoid(x)


def _wig_entries(vx, vy, vz):
    n = jnp.sqrt(vx * vx + vy * vy + vz * vz)
    inv = 1.0 / (n + 1e-12)
    hx, hy, hz = vx * inv, vy * inv, vz * inv
    near = jnp.abs(hz) > 0.99
    rx = jnp.where(near, 1.0, 0.0)
    rz = jnp.where(near, 0.0, 1.0)
    ax_ = hy * rz
    ay_ = hz * rx - hx * rz
    az_ = -hy * rx
    an = jnp.sqrt(ax_ * ax_ + ay_ * ay_ + az_ * az_)
    ainv = 1.0 / (an + 1e-12)
    ax, ay, az = ax_ * ainv, ay_ * ainv, az_ * ainv
    cx = ay * hz - az * hy
    cy = az * hx - ax * hz
    cz = ax * hy - ay * hx
    R = ((ax, ay, az), (hx, hy, hz), (cx, cy, cz))
    p = (1, 2, 0)
    D1 = [R[p[i]][p[l]] for i in range(3) for l in range(3)]
    prod = {}

    def rr(a_, b_, d_, c_):
        key = (a_, b_, d_, c_)
        if key not in prod:
            prod[key] = R[a_][b_] * R[d_][c_]
        return prod[key]

    D2 = []
    for nn in range(5):
        for mm in range(5):
            acc = None
            for (aa, dd), bv in _B2_NZ[nn]:
                for (bb, cc), bv2 in _B2_NZ[mm]:
                    term = (bv * bv2) * rr(aa, bb, dd, cc)
                    acc = term if acc is None else acc + term
            D2.append(acc)
    d = jnp.sqrt(n * n + 1e-24)
    xq = d * (1.0 / _CUTOFF)
    x5 = xq * xq * xq * xq * xq
    env = jnp.where(xq < 1.0, 1.0 - 21.0 * x5 + 35.0 * x5 * xq - 15.0 * x5 * xq * xq, 0.0)
    return D1 + D2 + [env, d]


def _edge_a_kernel(evt_ref, se_ref, re_ref, Wed0_ref, bed0_ref, Wed1_ref,
                   bed1_ref, Wed2_ref, radW0_ref, radb0_ref, radW1_ref,
                   radb1_ref, xg_ref, wig_ref, rad_ref):
    B = se_ref.shape[0]
    vx = evt_ref[0:1, :]
    vy = evt_ref[1:2, :]
    vz = evt_ref[2:3, :]
    rows = _wig_entries(vx, vy, vz)
    rows.extend([jnp.zeros((1, B), jnp.float32)] * 4)
    wig_t = jnp.concatenate(rows, axis=0)
    wig_ref[...] = wig_t
    wig_bt = wig_t.T
    d_b1 = wig_bt[:, 35:36]
    env_b1 = wig_bt[:, 34:35]
    offs = jax.lax.broadcasted_iota(jnp.int32, (1, _NUM_RBF), 1).astype(jnp.float32) * _DELTA
    rbf = jnp.exp(_COEFF * (d_b1 - offs) ** 2)
    ee = jnp.concatenate([rbf, se_ref[:, :64], re_ref[:, 64:128]], axis=1)
    h = _silu(jnp.dot(ee, Wed0_ref[...], preferred_element_type=jnp.float32) + bed0_ref[...])
    h = _silu(jnp.dot(h, Wed1_ref[...], preferred_element_type=jnp.float32) + bed1_ref[...])
    h3 = jnp.dot(h, Wed2_ref[...], preferred_element_type=jnp.float32)
    rad = jnp.dot(_silu(jnp.dot(ee, radW0_ref[...], preferred_element_type=jnp.float32) + radb0_ref[...]),
                  radW1_ref[...], preferred_element_type=jnp.float32) + radb1_ref[...]
    rad_ref[...] = rad.T
    ha = h3[:, 0:64]
    hb = h3[:, 64:128]
    hc = h3[:, 128:192]
    envb = jnp.broadcast_to(env_b1, (B, 64))
    pieces = [ha * envb]
    for l in range(3):
        wbk = jnp.broadcast_to(wig_bt[:, 3 + l:4 + l], (B, 64))
        pieces.append(wbk * envb * hb)
    for m in range(5):
        wbk = jnp.broadcast_to(wig_bt[:, 19 + m:20 + m], (B, 64))
        pieces.append(wbk * envb * hc)
    xg = jnp.concatenate(pieces, axis=1)
    xg_ref[...] = jnp.concatenate([xg, jnp.zeros((B, 64), jnp.float32)], axis=1)


def _sh_ln_flat(nf, g_ref, b_ref, eps=1e-5):
    x0 = nf[:, :64]
    mu = jnp.mean(x0, axis=1, keepdims=True)
    var = jnp.mean((x0 - mu) ** 2, axis=1, keepdims=True)
    y0 = (x0 - mu) * jax.lax.rsqrt(var + eps) * g_ref[0:1, :] + b_ref[...]
    x1 = nf[:, 64:256]
    r1 = jax.lax.rsqrt(jnp.mean(x1 * x1, axis=1, keepdims=True) + eps)
    y1 = x1 * r1 * jnp.tile(g_ref[1:2, :], (1, 3))
    x2 = nf[:, 256:576]
    r2 = jax.lax.rsqrt(jnp.mean(x2 * x2, axis=1, keepdims=True) + eps)
    y2 = x2 * r2 * jnp.tile(g_ref[2:3, :], (1, 5))
    return jnp.concatenate([y0, y1, y2], axis=1)


def _node_b_kernel(nf0_ref, s1_ref, g_ref, b_ref, res_ref, x_ref):
    s = s1_ref[:, :576] * 0.2
    nf = jnp.concatenate([nf0_ref[...] + s[:, :64], s[:, 64:]], axis=1)
    res_ref[...] = nf
    xln = _sh_ln_flat(nf, g_ref, b_ref)
    xr16 = xln.astype(jnp.bfloat16).astype(jnp.float32)
    u = jax.lax.bitcast_convert_type(xr16, jnp.uint32)
    hi = u[:, 0:288]
    lo = u[:, 288:576]
    packed = hi | (lo >> 16)
    packed = jnp.concatenate(
        [packed, jnp.zeros((xln.shape[0], 96), jnp.uint32)], axis=1)
    x_ref[...] = jax.lax.bitcast_convert_type(packed, jnp.float32)


def _apply_wig_t(wigt, x9, inv):
    out = [x9[0]]
    for i in range(3):
        acc = None
        for j in range(3):
            k = (3 * j + i) if inv else (3 * i + j)
            t = wigt[k:k + 1, :] * x9[1 + j]
            acc = t if acc is None else acc + t
        out.append(acc)
    for nn in range(5):
        acc = None
        for mm in range(5):
            k = 9 + (5 * mm + nn if inv else 5 * nn + mm)
            t = wigt[k:k + 1, :] * x9[4 + mm]
            acc = t if acc is None else acc + t
        out.append(acc)
    return out


def _edge_c_kernel(xs_ref, xr_ref, wig_ref, rad_ref, W1t_ref, Wgt_ref,
                   W2t_ref, out_ref):
    B = xs_ref.shape[0]
    wigt = wig_ref[...]
    def unpack(ref):
        u = jax.lax.bitcast_convert_type(ref[:, 0:288], jnp.uint32)
        hi = jax.lax.bitcast_convert_type(u & jnp.uint32(0xFFFF0000),
                                          jnp.float32)
        lo = jax.lax.bitcast_convert_type(u << 16, jnp.float32)
        return jnp.concatenate([hi, lo], axis=1)

    xsT = unpack(xs_ref).T
    xrT = unpack(xr_ref).T
    xs9 = [xsT[64 * i:64 * (i + 1), :] for i in range(9)]
    xr9 = [xrT[64 * i:64 * (i + 1), :] for i in range(9)]
    xs = _apply_wig_t(wigt, xs9, inv=False)
    xr = _apply_wig_t(wigt, xr9, inv=False)
    radt = rad_ref[...]
    cols = [jnp.concatenate([xs[i], xr[i]], axis=0) * radt for i in range(9)]
    msg = jnp.concatenate(cols, axis=1)
    m1 = jnp.dot(W1t_ref[...], msg, preferred_element_type=jnp.float32)
    s = m1[:, 0:B]
    g = jax.nn.sigmoid(jnp.dot(Wgt_ref[...], s, preferred_element_type=jnp.float32))
    m2 = jnp.concatenate([_silu(s), m1[:, B:] * jnp.tile(g, (1, 8))], axis=1)
    m3 = jnp.dot(W2t_ref[...], m2, preferred_element_type=jnp.float32)
    m9 = [m3[:, i * B:(i + 1) * B] for i in range(9)]
    outp = _apply_wig_t(wigt, m9, inv=True)
    env = wigt[34:35, :]
    mgT = jnp.concatenate([o * env for o in outp], axis=0)
    mg = mgT.T
    out_ref[...] = jnp.concatenate([mg, jnp.zeros((B, 64), jnp.float32)], axis=1)


def _node_d_kernel(res_ref, s2_ref, ln2g_ref, ln2b_ref, A1_ref, bA1_ref,
                   Ag_ref, A2_ref, bA2_ref, lnfg_ref, lnfb_ref, out_ref):
    Bn = res_ref.shape[0]
    nf = res_ref[...] + s2_ref[:, :576]
    x = _sh_ln_flat(nf, ln2g_ref, ln2b_ref)
    xs = jnp.concatenate([x[:, 64 * i:64 * (i + 1)] for i in range(9)], axis=0)
    hh = jnp.dot(xs, A1_ref[...], preferred_element_type=jnp.float32) + bA1_ref[...]
    s = hh[0:Bn]
    g = jax.nn.sigmoid(jnp.dot(s, Ag_ref[...], preferred_element_type=jnp.float32))
    hh2 = jnp.concatenate([_silu(s), hh[Bn:] * jnp.tile(g, (8, 1))], axis=0)
    hh3 = jnp.dot(hh2, A2_ref[...], preferred_element_type=jnp.float32) + bA2_ref[...]
    nf2 = nf + jnp.concatenate([hh3[i * Bn:(i + 1) * Bn] for i in range(9)], axis=1)
    out_ref[...] = _sh_ln_flat(nf2, lnfg_ref, lnfb_ref)


_W = 128
_NP = 10240
_NS = 5
_SPC = 3


def _sc_segsum(rows, idx2d, zeros, n_pad):
    E = rows.shape[0]
    n_win = E // _W
    kmax = (n_win + 15) // 16
    mesh = plsc.VectorSubcoreMesh(core_axis_name="c", subcore_axis_name="s")

    @functools.partial(
        pl.kernel,
        out_type=jax.ShapeDtypeStruct((n_pad, _NS * _W), jnp.float32),
        mesh=mesh,
        scratch_types=[
            pltpu.VMEM((_W,), jnp.int32),
            pltpu.VMEM((_W,), jnp.int32),
            pltpu.VMEM((_W, _W), jnp.float32),
            pltpu.VMEM((_W, _W), jnp.float32),
            pltpu.VMEM_SHARED((n_pad, _W), jnp.float32),
            pltpu.SemaphoreType.DMA,
            pltpu.SemaphoreType.DMA,
        ],
    )
    def seg(rows_hbm, idx_hbm, zeros_hbm, out_hbm, idx_v0, idx_v1, rows_v0,
            rows_v1, acc, sem0, sem1):
        cid = lax.axis_index("c")
        sid = lax.axis_index("s")
        stripe = n_pad // 16
        idx_b = (idx_v0, idx_v1)
        row_b = (rows_v0, rows_v1)
        sems = (sem0, sem1)
        for j in range(_SPC):
            strip = cid * _SPC + j

            @pl.when(strip < _NS)
            def _(strip=strip):
                pltpu.sync_copy(zeros_hbm,
                                acc.at[pl.ds(sid * stripe, stripe)])
                plsc.subcore_barrier()

                def cps(k, slot):
                    w = sid + 16 * k
                    return (
                        pltpu.make_async_copy(idx_hbm.at[w], idx_b[slot],
                                              sems[slot]),
                        pltpu.make_async_copy(
                            rows_hbm.at[pl.ds(w * _W, _W),
                                        pl.ds(strip * _W, _W)],
                            row_b[slot], sems[slot]),
                    )

                for s0 in range(2):
                    @pl.when(sid + 16 * s0 < n_win)
                    def _(s0=s0):
                        a, b = cps(s0, s0)
                        a.start()
                        b.start()

                @pl.loop(0, (kmax + 1) // 2)
                def _(kk):
                    for slot in range(2):
                        k = 2 * kk + slot
                        w = sid + 16 * k

                        @pl.when(w < n_win)
                        def _(k=k, slot=slot, w=w):
                            a, b = cps(k, slot)
                            a.wait()
                            b.wait()
                            pltpu.sync_copy(row_b[slot], acc.at[idx_b[slot]],
                                            add=True)

                            @pl.when(w + 32 < n_win)
                            def _():
                                a2, b2 = cps(k + 2, slot)
                                a2.start()
                                b2.start()

                plsc.subcore_barrier()
                pltpu.sync_copy(
                    acc.at[pl.ds(sid * stripe, stripe)],
                    out_hbm.at[pl.ds(sid * stripe, stripe),
                               pl.ds(strip * _W, _W)])
                plsc.subcore_barrier()

    return seg(rows, idx2d, zeros)


def _sc_gather2(table, idx_a2d, idx_b2d):
    D = table.shape[1]
    dt = table.dtype
    n_win = idx_a2d.shape[0]
    E = n_win * _W
    kmax = (n_win + 31) // 32
    mesh = plsc.VectorSubcoreMesh(core_axis_name="c", subcore_axis_name="s")

    @functools.partial(
        pl.kernel,
        out_type=[jax.ShapeDtypeStruct((E, D), dt),
                  jax.ShapeDtypeStruct((E, D), dt)],
        mesh=mesh,
        scratch_types=[
            pltpu.VMEM((_W,), jnp.int32),
            pltpu.VMEM((_W, D), dt),
        ],
    )
    def gat(tab_hbm, ia_hbm, ib_hbm, oa_hbm, ob_hbm, idx_v, rows_v):
        cid = lax.axis_index("c")
        sid = lax.axis_index("s")
        wid = cid * 16 + sid

        @pl.loop(0, kmax)
        def _(k):
            w = wid + 32 * k

            @pl.when(w < n_win)
            def _():
                for ih, oh in ((ia_hbm, oa_hbm), (ib_hbm, ob_hbm)):
                    pltpu.sync_copy(ih.at[w], idx_v)
                    pltpu.sync_copy(tab_hbm.at[idx_v], rows_v)
                    pltpu.sync_copy(rows_v, oh.at[pl.ds(w * _W, _W)])

    return gat(table, idx_a2d, idx_b2d)


def _species_kernel(sp_ref, tab_ref, csd_ref, nf0_ref, t1_ref):
    Bn = sp_ref.shape[2]
    sp = sp_ref[0].T
    lanes = jax.lax.broadcasted_iota(jnp.int32, (Bn, 128), 1)
    oh = jnp.where(lanes == sp, 1.0, 0.0)
    emb = jnp.dot(oh, tab_ref[...], preferred_element_type=jnp.float32)
    i = pl.program_id(0)
    per_sys = (pl.num_programs(0) * Bn) // 4
    row = jax.lax.broadcasted_iota(jnp.int32, (Bn, 4), 0) + i * Bn
    sysid = jax.lax.broadcasted_iota(jnp.int32, (1, 4), 1)
    ohsys = jnp.where(row // per_sys == sysid, 1.0, 0.0)
    csd = jnp.dot(ohsys, csd_ref[...], preferred_element_type=jnp.float32)
    nf0_ref[...] = emb[:, 0:64] + csd
    t1_ref[...] = emb[:, 64:192]


def _full(shape):
    return pl.BlockSpec(shape, lambda i: tuple(0 for _ in shape))


def kernel(edge_vectors, csd_mixed_emb, sphere_emb, senders_emb, receivers_emb,
           W_ed0, b_ed0, W_ed1, b_ed1, W_ed2, ln1_g, ln1_b, rad_W0, rad_b0,
           rad_W1, rad_b1, W1, Wg, W2, ln2_g, ln2_b, A1, bA1, Ag, A2, bA2,
           lnf_g, lnf_b, node_species, senders, receivers, n_node):
    N = node_species.shape[0]
    E = senders.shape[0]
    C = csd_mixed_emb.shape[1]
    f32 = jnp.float32

    Bn0 = 1000 if N % 1000 == 0 else N
    sp_tab = jnp.zeros((128, 192), f32)
    sp_tab = sp_tab.at[:sphere_emb.shape[0], 0:64].set(sphere_emb)
    sp_tab = sp_tab.at[:sphere_emb.shape[0], 64:128].set(senders_emb)
    sp_tab = sp_tab.at[:sphere_emb.shape[0], 128:192].set(receivers_emb)
    nf0, T1 = pl.pallas_call(
        _species_kernel,
        grid=(N // Bn0,),
        in_specs=[
            pl.BlockSpec((1, 1, Bn0), lambda i: (i, 0, 0)),
            _full((128, 192)),
            _full(csd_mixed_emb.shape),
        ],
        out_specs=[
            pl.BlockSpec((Bn0, 64), lambda i: (i, 0)),
            pl.BlockSpec((Bn0, 128), lambda i: (i, 0)),
        ],
        out_shape=[
            jax.ShapeDtypeStruct((N, 64), f32),
            jax.ShapeDtypeStruct((N, 128), f32),
        ],
    )(node_species.astype(jnp.int32).reshape(N // Bn0, 1, Bn0), sp_tab,
      csd_mixed_emb)

    B = 640 if E % 640 == 0 else E
    Bn = 1000 if N % 1000 == 0 else N
    n_win = E // _W
    senders2d = senders.astype(jnp.int32).reshape(n_win, _W)
    receivers2d = receivers.astype(jnp.int32).reshape(n_win, _W)
    se, re = _sc_gather2(T1, senders2d, receivers2d)
    evt = edge_vectors.T
    b_ed0_2 = b_ed0.reshape(1, -1)
    b_ed1_2 = b_ed1.reshape(1, -1)
    rad_b0_2 = rad_b0.reshape(1, -1)
    rad_b1_2 = rad_b1.reshape(1, -1)

    xglob, wigc, rad = pl.pallas_call(
        _edge_a_kernel,
        grid=(E // B,),
        in_specs=[
            pl.BlockSpec((3, B), lambda i: (0, i)),
            pl.BlockSpec((B, 128), lambda i: (i, 0)),
            pl.BlockSpec((B, 128), lambda i: (i, 0)),
            _full(W_ed0.shape), _full((1, 64)),
            _full(W_ed1.shape), _full((1, 64)),
            _full(W_ed2.shape),
            _full(rad_W0.shape), _full((1, 64)),
            _full(rad_W1.shape), _full((1, 128)),
        ],
        out_specs=[
            pl.BlockSpec((B, 640), lambda i: (i, 0)),
            pl.BlockSpec((40, B), lambda i: (0, i)),
            pl.BlockSpec((128, B), lambda i: (0, i)),
        ],
        out_shape=[
            jax.ShapeDtypeStruct((E, 640), f32),
            jax.ShapeDtypeStruct((40, E), f32),
            jax.ShapeDtypeStruct((128, E), f32),
        ],
    )(evt, se, re, W_ed0, b_ed0_2, W_ed1, b_ed1_2, W_ed2, rad_W0, rad_b0_2,
      rad_W1, rad_b1_2)

    idx2d = receivers2d
    sc_zeros = jnp.zeros((_NP // 16, _W), jnp.float32)

    s1 = _sc_segsum(xglob, idx2d, sc_zeros, _NP)[:N]

    lnb1 = ln1_b.reshape(1, -1)
    res, x = pl.pallas_call(
        _node_b_kernel,
        grid=(N // Bn,),
        in_specs=[
            pl.BlockSpec((Bn, 64), lambda i: (i, 0)),
            pl.BlockSpec((Bn, 640), lambda i: (i, 0)),
            _full(ln1_g.shape), _full((1, 64)),
        ],
        out_specs=[
            pl.BlockSpec((Bn, 576), lambda i: (i, 0)),
            pl.BlockSpec((Bn, 384), lambda i: (i, 0)),
        ],
        out_shape=[
            jax.ShapeDtypeStruct((N, 576), f32),
            jax.ShapeDtypeStruct((N, 384), f32),
        ],
    )(nf0, s1, ln1_g, lnb1)

    xs_raw, xr_raw = _sc_gather2(x, senders2d, receivers2d)

    msgglob = pl.pallas_call(
        _edge_c_kernel,
        grid=(E // B,),
        in_specs=[
            pl.BlockSpec((B, 384), lambda i: (i, 0)),
            pl.BlockSpec((B, 384), lambda i: (i, 0)),
            pl.BlockSpec((40, B), lambda i: (0, i)),
            pl.BlockSpec((128, B), lambda i: (0, i)),
            _full((64, 128)), _full(Wg.shape), _full(W2.shape),
        ],
        out_specs=pl.BlockSpec((B, 640), lambda i: (i, 0)),
        out_shape=jax.ShapeDtypeStruct((E, 640), f32),
    )(xs_raw, xr_raw, wigc, rad, W1.T, Wg.T, W2.T)

    s2 = _sc_segsum(msgglob, idx2d, sc_zeros, _NP)[:N]

    out = pl.pallas_call(
        _node_d_kernel,
        grid=(N // Bn,),
        in_specs=[
            pl.BlockSpec((Bn, 576), lambda i: (i, 0)),
            pl.BlockSpec((Bn, 640), lambda i: (i, 0)),
            _full(ln2_g.shape), _full((1, 64)),
            _full(A1.shape), _full((1, 64)),
            _full(Ag.shape),
            _full(A2.shape), _full((1, 64)),
            _full(lnf_g.shape), _full((1, 64)),
        ],
        out_specs=pl.BlockSpec((Bn, 576), lambda i: (i, 0)),
        out_shape=jax.ShapeDtypeStruct((N, 576), f32),
    )(res, s2, ln2_g, ln2_b.reshape(1, -1), A1, bA1.reshape(1, -1), Ag, A2,
      bA2.reshape(1, -1), lnf_g, lnf_b.reshape(1, -1))

    return out.reshape(N, 9, C)


# speedup vs baseline: 1.0466x; 1.0466x over previous
import functools

import numpy as np
import jax
import jax.numpy as jnp
from jax import lax
from jax.experimental import pallas as pl
from jax.experimental.pallas import tpu as pltpu
from jax.experimental.pallas import tpu_sc as plsc

_NUM_RBF = 128
_CUTOFF = 5.0
_DELTA = _CUTOFF / (_NUM_RBF - 1)
_COEFF = -0.5 / (2.0 * _DELTA) ** 2
_S2 = float(1.0 / np.sqrt(2.0))
_S6 = float(1.0 / np.sqrt(6.0))

_B2_NZ = (
    (((0, 1), _S2), ((1, 0), _S2)),
    (((1, 2), _S2), ((2, 1), _S2)),
    (((0, 0), -_S6), ((1, 1), -_S6), ((2, 2), 2.0 * _S6)),
    (((0, 2), _S2), ((2, 0), _S2)),
    (((0, 0), _S2), ((1, 1), -_S2)),
)


def _silu(x):
    return x * jax.nn.sigmoid(x)


def _wig_entries(vx, vy, vz):
    n = jnp.sqrt(vx * vx + vy * vy + vz * vz)
    inv = 1.0 / (n + 1e-12)
    hx, hy, hz = vx * inv, vy * inv, vz * inv
    near = jnp.abs(hz) > 0.99
    rx = jnp.where(near, 1.0, 0.0)
    rz = jnp.where(near, 0.0, 1.0)
    ax_ = hy * rz
    ay_ = hz * rx - hx * rz
    az_ = -hy * rx
    an = jnp.sqrt(ax_ * ax_ + ay_ * ay_ + az_ * az_)
    ainv = 1.0 / (an + 1e-12)
    ax, ay, az = ax_ * ainv, ay_ * ainv, az_ * ainv
    cx = ay * hz - az * hy
    cy = az * hx - ax * hz
    cz = ax * hy - ay * hx
    R = ((ax, ay, az), (hx, hy, hz), (cx, cy, cz))
    p = (1, 2, 0)
    D1 = [R[p[i]][p[l]] for i in range(3) for l in range(3)]
    prod = {}

    def rr(a_, b_, d_, c_):
        key = (a_, b_, d_, c_)
        if key not in prod:
            prod[key] = R[a_][b_] * R[d_][c_]
        return prod[key]

    D2 = []
    for nn in range(5):
        for mm in range(5):
            acc = None
            for (aa, dd), bv in _B2_NZ[nn]:
                for (bb, cc), bv2 in _B2_NZ[mm]:
                    term = (bv * bv2) * rr(aa, bb, dd, cc)
                    acc = term if acc is None else acc + term
            D2.append(acc)
    d = jnp.sqrt(n * n + 1e-24)
    xq = d * (1.0 / _CUTOFF)
    x5 = xq * xq * xq * xq * xq
    env = jnp.where(xq < 1.0, 1.0 - 21.0 * x5 + 35.0 * x5 * xq - 15.0 * x5 * xq * xq, 0.0)
    return D1 + D2 + [env, d]


def _edge_a_kernel(evt_ref, se_ref, re_ref, Wed0_ref, bed0_ref, Wed1_ref,
                   bed1_ref, Wed2_ref, radW0_ref, radb0_ref, radW1_ref,
                   radb1_ref, xg_ref, wig_ref, rad_ref):
    B = se_ref.shape[0]
    vx = evt_ref[0:1, :]
    vy = evt_ref[1:2, :]
    vz = evt_ref[2:3, :]
    rows = _wig_entries(vx, vy, vz)
    rows.extend([jnp.zeros((1, B), jnp.float32)] * 4)
    wig_t = jnp.concatenate(rows, axis=0)
    wig_ref[...] = wig_t
    wig_bt = wig_t.T
    d_b1 = wig_bt[:, 35:36]
    env_b1 = wig_bt[:, 34:35]
    offs = jax.lax.broadcasted_iota(jnp.int32, (1, _NUM_RBF), 1).astype(jnp.float32) * _DELTA
    rbf = jnp.exp(_COEFF * (d_b1 - offs) ** 2)
    ee = jnp.concatenate([rbf, se_ref[:, :64], re_ref[:, 64:128]], axis=1)
    h = _silu(jnp.dot(ee, Wed0_ref[...], preferred_element_type=jnp.float32) + bed0_ref[...])
    h = _silu(jnp.dot(h, Wed1_ref[...], preferred_element_type=jnp.float32) + bed1_ref[...])
    h3 = jnp.dot(h, Wed2_ref[...], preferred_element_type=jnp.float32)
    rad = jnp.dot(_silu(jnp.dot(ee, radW0_ref[...], preferred_element_type=jnp.float32) + radb0_ref[...]),
                  radW1_ref[...], preferred_element_type=jnp.float32) + radb1_ref[...]
    rad_ref[...] = rad.T
    ha = h3[:, 0:64]
    hb = h3[:, 64:128]
    hc = h3[:, 128:192]
    envb = jnp.broadcast_to(env_b1, (B, 64))
    pieces = [ha * envb]
    for l in range(3):
        wbk = jnp.broadcast_to(wig_bt[:, 3 + l:4 + l], (B, 64))
        pieces.append(wbk * envb * hb)
    for m in range(5):
        wbk = jnp.broadcast_to(wig_bt[:, 19 + m:20 + m], (B, 64))
        pieces.append(wbk * envb * hc)
    xg = jnp.concatenate(pieces, axis=1)
    xg_ref[...] = jnp.concatenate([xg, jnp.zeros((B, 64), jnp.float32)], axis=1)


def _sh_ln_flat(nf, g_ref, b_ref, eps=1e-5):
    x0 = nf[:, :64]
    mu = jnp.mean(x0, axis=1, keepdims=True)
    var = jnp.mean((x0 - mu) ** 2, axis=1, keepdims=True)
    y0 = (x0 - mu) * jax.lax.rsqrt(var + eps) * g_ref[0:1, :] + b_ref[...]
    x1 = nf[:, 64:256]
    r1 = jax.lax.rsqrt(jnp.mean(x1 * x1, axis=1, keepdims=True) + eps)
    y1 = x1 * r1 * jnp.tile(g_ref[1:2, :], (1, 3))
    x2 = nf[:, 256:576]
    r2 = jax.lax.rsqrt(jnp.mean(x2 * x2, axis=1, keepdims=True) + eps)
    y2 = x2 * r2 * jnp.tile(g_ref[2:3, :], (1, 5))
    return jnp.concatenate([y0, y1, y2], axis=1)


def _node_b_kernel(nf0_ref, s1_ref, g_ref, b_ref, res_ref, x_ref):
    s = s1_ref[:, :576] * 0.2
    nf = jnp.concatenate([nf0_ref[...] + s[:, :64], s[:, 64:]], axis=1)
    res_ref[...] = nf
    xln = _sh_ln_flat(nf, g_ref, b_ref)
    xr16 = xln.astype(jnp.bfloat16).astype(jnp.float32)
    u = jax.lax.bitcast_convert_type(xr16, jnp.uint32)
    hi = u[:, 0:288]
    lo = u[:, 288:576]
    packed = hi | (lo >> 16)
    packed = jnp.concatenate(
        [packed, jnp.zeros((xln.shape[0], 96), jnp.uint32)], axis=1)
    x_ref[...] = jax.lax.bitcast_convert_type(packed, jnp.float32)


def _apply_wig_t(wigt, x9, inv):
    out = [x9[0]]
    for i in range(3):
        acc = None
        for j in range(3):
            k = (3 * j + i) if inv else (3 * i + j)
            t = wigt[k:k + 1, :] * x9[1 + j]
            acc = t if acc is None else acc + t
        out.append(acc)
    for nn in range(5):
        acc = None
        for mm in range(5):
            k = 9 + (5 * mm + nn if inv else 5 * nn + mm)
            t = wigt[k:k + 1, :] * x9[4 + mm]
            acc = t if acc is None else acc + t
        out.append(acc)
    return out


def _edge_c_kernel(xs_ref, xr_ref, wig_ref, rad_ref, W1t_ref, Wgt_ref,
                   W2t_ref, out_ref):
    B = xs_ref.shape[0]
    wigt = wig_ref[...]
    def unpack(ref):
        u = jax.lax.bitcast_convert_type(ref[:, 0:288], jnp.uint32)
        hi = jax.lax.bitcast_convert_type(u & jnp.uint32(0xFFFF0000),
                                          jnp.float32)
        lo = jax.lax.bitcast_convert_type(u << 16, jnp.float32)
        return jnp.concatenate([hi, lo], axis=1)

    xsT = unpack(xs_ref).T
    xrT = unpack(xr_ref).T
    xs9 = [xsT[64 * i:64 * (i + 1), :] for i in range(9)]
    xr9 = [xrT[64 * i:64 * (i + 1), :] for i in range(9)]
    xs = _apply_wig_t(wigt, xs9, inv=False)
    xr = _apply_wig_t(wigt, xr9, inv=False)
    radt = rad_ref[...]
    cols = [jnp.concatenate([xs[i], xr[i]], axis=0) * radt for i in range(9)]
    msg = jnp.concatenate(cols, axis=1)
    m1 = jnp.dot(W1t_ref[...], msg, preferred_element_type=jnp.float32)
    s = m1[:, 0:B]
    g = jax.nn.sigmoid(jnp.dot(Wgt_ref[...], s, preferred_element_type=jnp.float32))
    m2 = jnp.concatenate([_silu(s), m1[:, B:] * jnp.tile(g, (1, 8))], axis=1)
    m3 = jnp.dot(W2t_ref[...], m2, preferred_element_type=jnp.float32)
    m9 = [m3[:, i * B:(i + 1) * B] for i in range(9)]
    outp = _apply_wig_t(wigt, m9, inv=True)
    env = wigt[34:35, :]
    mgT = jnp.concatenate([o * env for o in outp], axis=0)
    mg = mgT.T
    out_ref[...] = jnp.concatenate([mg, jnp.zeros((B, 64), jnp.float32)], axis=1)


def _node_d_kernel(res_ref, s2_ref, ln2g_ref, ln2b_ref, A1_ref, bA1_ref,
                   Ag_ref, A2_ref, bA2_ref, lnfg_ref, lnfb_ref, out_ref):
    Bn = res_ref.shape[0]
    nf = res_ref[...] + s2_ref[:, :576]
    x = _sh_ln_flat(nf, ln2g_ref, ln2b_ref)
    xs = jnp.concatenate([x[:, 64 * i:64 * (i + 1)] for i in range(9)], axis=0)
    hh = jnp.dot(xs, A1_ref[...], preferred_element_type=jnp.float32) + bA1_ref[...]
    s = hh[0:Bn]
    g = jax.nn.sigmoid(jnp.dot(s, Ag_ref[...], preferred_element_type=jnp.float32))
    hh2 = jnp.concatenate([_silu(s), hh[Bn:] * jnp.tile(g, (8, 1))], axis=0)
    hh3 = jnp.dot(hh2, A2_ref[...], preferred_element_type=jnp.float32) + bA2_ref[...]
    nf2 = nf + jnp.concatenate([hh3[i * Bn:(i + 1) * Bn] for i in range(9)], axis=1)
    out_ref[...] = _sh_ln_flat(nf2, lnfg_ref, lnfb_ref)


_W = 128
_NP = 10240
_NS = 5
_SPC = 3


def _sc_segsum(rows, idx2d, zeros, n_pad):
    E = rows.shape[0]
    n_win = E // _W
    kmax = (n_win + 15) // 16
    mesh = plsc.VectorSubcoreMesh(core_axis_name="c", subcore_axis_name="s")

    @functools.partial(
        pl.kernel,
        out_type=jax.ShapeDtypeStruct((n_pad, _NS * _W), jnp.float32),
        mesh=mesh,
        scratch_types=[
            pltpu.VMEM((_W,), jnp.int32),
            pltpu.VMEM((_W,), jnp.int32),
            pltpu.VMEM((_W, _W), jnp.float32),
            pltpu.VMEM((_W, _W), jnp.float32),
            pltpu.VMEM_SHARED((n_pad, _W), jnp.float32),
            pltpu.SemaphoreType.DMA,
            pltpu.SemaphoreType.DMA,
        ],
    )
    def seg(rows_hbm, idx_hbm, zeros_hbm, out_hbm, idx_v0, idx_v1, rows_v0,
            rows_v1, acc, sem0, sem1):
        cid = lax.axis_index("c")
        sid = lax.axis_index("s")
        stripe = n_pad // 16
        idx_b = (idx_v0, idx_v1)
        row_b = (rows_v0, rows_v1)
        sems = (sem0, sem1)
        for j in range(_SPC):
            strip = cid * _SPC + j

            @pl.when(strip < _NS)
            def _(strip=strip):
                pltpu.sync_copy(zeros_hbm,
                                acc.at[pl.ds(sid * stripe, stripe)])
                plsc.subcore_barrier()

                def cps(k, slot):
                    w = sid + 16 * k
                    return (
                        pltpu.make_async_copy(idx_hbm.at[w], idx_b[slot],
                                              sems[slot]),
                        pltpu.make_async_copy(
                            rows_hbm.at[pl.ds(w * _W, _W),
                                        pl.ds(strip * _W, _W)],
                            row_b[slot], sems[slot]),
                    )

                for s0 in range(2):
                    @pl.when(sid + 16 * s0 < n_win)
                    def _(s0=s0):
                        a, b = cps(s0, s0)
                        a.start()
                        b.start()

                @pl.loop(0, (kmax + 1) // 2)
                def _(kk):
                    for slot in range(2):
                        k = 2 * kk + slot
                        w = sid + 16 * k

                        @pl.when(w < n_win)
                        def _(k=k, slot=slot, w=w):
                            a, b = cps(k, slot)
                            a.wait()
                            b.wait()
                            pltpu.sync_copy(row_b[slot], acc.at[idx_b[slot]],
                                            add=True)

                            @pl.when(w + 32 < n_win)
                            def _():
                                a2, b2 = cps(k + 2, slot)
                                a2.start()
                                b2.start()

                plsc.subcore_barrier()
                pltpu.sync_copy(
                    acc.at[pl.ds(sid * stripe, stripe)],
                    out_hbm.at[pl.ds(sid * stripe, stripe),
                               pl.ds(strip * _W, _W)])
                plsc.subcore_barrier()

    return seg(rows, idx2d, zeros)


def _sc_gather2(table, idx_a2d, idx_b2d):
    D = table.shape[1]
    dt = table.dtype
    n_win = idx_a2d.shape[0]
    E = n_win * _W
    kmax = (n_win + 31) // 32
    mesh = plsc.VectorSubcoreMesh(core_axis_name="c", subcore_axis_name="s")

    @functools.partial(
        pl.kernel,
        out_type=[jax.ShapeDtypeStruct((E, D), dt),
                  jax.ShapeDtypeStruct((E, D), dt)],
        mesh=mesh,
        scratch_types=[
            pltpu.VMEM((_W,), jnp.int32),
            pltpu.VMEM((_W,), jnp.int32),
            pltpu.VMEM((_W, D), dt),
            pltpu.VMEM((_W, D), dt),
            pltpu.SemaphoreType.DMA,
            pltpu.SemaphoreType.DMA,
            pltpu.SemaphoreType.DMA,
            pltpu.SemaphoreType.DMA,
        ],
    )
    def gat(tab_hbm, ia_hbm, ib_hbm, oa_hbm, ob_hbm, ia_v, ib_v, ra_v, rb_v,
            sia, sib, sga, sgb):
        cid = lax.axis_index("c")
        sid = lax.axis_index("s")
        wid = cid * 16 + sid

        @pl.when(wid < n_win)
        def _():
            pltpu.make_async_copy(ia_hbm.at[wid], ia_v, sia).start()
            pltpu.make_async_copy(ib_hbm.at[wid], ib_v, sib).start()

        @pl.loop(0, kmax)
        def _(k):
            w = wid + 32 * k

            @pl.when(w < n_win)
            def _():
                pltpu.make_async_copy(ia_hbm.at[w], ia_v, sia).wait()
                pltpu.make_async_copy(ib_hbm.at[w], ib_v, sib).wait()
                ga = pltpu.make_async_copy(tab_hbm.at[ia_v], ra_v, sga)
                gb = pltpu.make_async_copy(tab_hbm.at[ib_v], rb_v, sgb)
                ga.start()
                gb.start()
                ga.wait()
                pltpu.sync_copy(ra_v, oa_hbm.at[pl.ds(w * _W, _W)])
                gb.wait()
                pltpu.sync_copy(rb_v, ob_hbm.at[pl.ds(w * _W, _W)])

                @pl.when(w + 32 < n_win)
                def _():
                    pltpu.make_async_copy(ia_hbm.at[w + 32], ia_v, sia).start()
                    pltpu.make_async_copy(ib_hbm.at[w + 32], ib_v, sib).start()

    return gat(table, idx_a2d, idx_b2d)


def _species_kernel(sp_ref, tab_ref, csd_ref, nf0_ref, t1_ref):
    Bn = sp_ref.shape[2]
    sp = sp_ref[0].T
    lanes = jax.lax.broadcasted_iota(jnp.int32, (Bn, 128), 1)
    oh = jnp.where(lanes == sp, 1.0, 0.0)
    emb = jnp.dot(oh, tab_ref[...], preferred_element_type=jnp.float32)
    i = pl.program_id(0)
    per_sys = (pl.num_programs(0) * Bn) // 4
    row = jax.lax.broadcasted_iota(jnp.int32, (Bn, 4), 0) + i * Bn
    sysid = jax.lax.broadcasted_iota(jnp.int32, (1, 4), 1)
    ohsys = jnp.where(row // per_sys == sysid, 1.0, 0.0)
    csd = jnp.dot(ohsys, csd_ref[...], preferred_element_type=jnp.float32)
    nf0_ref[...] = emb[:, 0:64] + csd
    t1_ref[...] = emb[:, 64:192]


def _full(shape):
    return pl.BlockSpec(shape, lambda i: tuple(0 for _ in shape))


def kernel(edge_vectors, csd_mixed_emb, sphere_emb, senders_emb, receivers_emb,
           W_ed0, b_ed0, W_ed1, b_ed1, W_ed2, ln1_g, ln1_b, rad_W0, rad_b0,
           rad_W1, rad_b1, W1, Wg, W2, ln2_g, ln2_b, A1, bA1, Ag, A2, bA2,
           lnf_g, lnf_b, node_species, senders, receivers, n_node):
    N = node_species.shape[0]
    E = senders.shape[0]
    C = csd_mixed_emb.shape[1]
    f32 = jnp.float32

    Bn0 = 1000 if N % 1000 == 0 else N
    sp_tab = jnp.zeros((128, 192), f32)
    sp_tab = sp_tab.at[:sphere_emb.shape[0], 0:64].set(sphere_emb)
    sp_tab = sp_tab.at[:sphere_emb.shape[0], 64:128].set(senders_emb)
    sp_tab = sp_tab.at[:sphere_emb.shape[0], 128:192].set(receivers_emb)
    nf0, T1 = pl.pallas_call(
        _species_kernel,
        grid=(N // Bn0,),
        in_specs=[
            pl.BlockSpec((1, 1, Bn0), lambda i: (i, 0, 0)),
            _full((128, 192)),
            _full(csd_mixed_emb.shape),
        ],
        out_specs=[
            pl.BlockSpec((Bn0, 64), lambda i: (i, 0)),
            pl.BlockSpec((Bn0, 128), lambda i: (i, 0)),
        ],
        out_shape=[
            jax.ShapeDtypeStruct((N, 64), f32),
            jax.ShapeDtypeStruct((N, 128), f32),
        ],
    )(node_species.astype(jnp.int32).reshape(N // Bn0, 1, Bn0), sp_tab,
      csd_mixed_emb)

    B = 640 if E % 640 == 0 else E
    Bn = 1000 if N % 1000 == 0 else N
    n_win = E // _W
    senders2d = senders.astype(jnp.int32).reshape(n_win, _W)
    receivers2d = receivers.astype(jnp.int32).reshape(n_win, _W)
    se, re = _sc_gather2(T1, senders2d, receivers2d)
    evt = edge_vectors.T
    b_ed0_2 = b_ed0.reshape(1, -1)
    b_ed1_2 = b_ed1.reshape(1, -1)
    rad_b0_2 = rad_b0.reshape(1, -1)
    rad_b1_2 = rad_b1.reshape(1, -1)

    xglob, wigc, rad = pl.pallas_call(
        _edge_a_kernel,
        grid=(E // B,),
        in_specs=[
            pl.BlockSpec((3, B), lambda i: (0, i)),
            pl.BlockSpec((B, 128), lambda i: (i, 0)),
            pl.BlockSpec((B, 128), lambda i: (i, 0)),
            _full(W_ed0.shape), _full((1, 64)),
            _full(W_ed1.shape), _full((1, 64)),
            _full(W_ed2.shape),
            _full(rad_W0.shape), _full((1, 64)),
            _full(rad_W1.shape), _full((1, 128)),
        ],
        out_specs=[
            pl.BlockSpec((B, 640), lambda i: (i, 0)),
            pl.BlockSpec((40, B), lambda i: (0, i)),
            pl.BlockSpec((128, B), lambda i: (0, i)),
        ],
        out_shape=[
            jax.ShapeDtypeStruct((E, 640), f32),
            jax.ShapeDtypeStruct((40, E), f32),
            jax.ShapeDtypeStruct((128, E), f32),
        ],
    )(evt, se, re, W_ed0, b_ed0_2, W_ed1, b_ed1_2, W_ed2, rad_W0, rad_b0_2,
      rad_W1, rad_b1_2)

    idx2d = receivers2d
    sc_zeros = jnp.zeros((_NP // 16, _W), jnp.float32)

    s1 = _sc_segsum(xglob, idx2d, sc_zeros, _NP)[:N]

    lnb1 = ln1_b.reshape(1, -1)
    res, x = pl.pallas_call(
        _node_b_kernel,
        grid=(N // Bn,),
        in_specs=[
            pl.BlockSpec((Bn, 64), lambda i: (i, 0)),
            pl.BlockSpec((Bn, 640), lambda i: (i, 0)),
            _full(ln1_g.shape), _full((1, 64)),
        ],
        out_specs=[
            pl.BlockSpec((Bn, 576), lambda i: (i, 0)),
            pl.BlockSpec((Bn, 384), lambda i: (i, 0)),
        ],
        out_shape=[
            jax.ShapeDtypeStruct((N, 576), f32),
            jax.ShapeDtypeStruct((N, 384), f32),
        ],
    )(nf0, s1, ln1_g, lnb1)

    xs_raw, xr_raw = _sc_gather2(x, senders2d, receivers2d)

    msgglob = pl.pallas_call(
        _edge_c_kernel,
        grid=(E // B,),
        in_specs=[
            pl.BlockSpec((B, 384), lambda i: (i, 0)),
            pl.BlockSpec((B, 384), lambda i: (i, 0)),
            pl.BlockSpec((40, B), lambda i: (0, i)),
            pl.BlockSpec((128, B), lambda i: (0, i)),
            _full((64, 128)), _full(Wg.shape), _full(W2.shape),
        ],
        out_specs=pl.BlockSpec((B, 640), lambda i: (i, 0)),
        out_shape=jax.ShapeDtypeStruct((E, 640), f32),
    )(xs_raw, xr_raw, wigc, rad, W1.T, Wg.T, W2.T)

    s2 = _sc_segsum(msgglob, idx2d, sc_zeros, _NP)[:N]

    out = pl.pallas_call(
        _node_d_kernel,
        grid=(N // Bn,),
        in_specs=[
            pl.BlockSpec((Bn, 576), lambda i: (i, 0)),
            pl.BlockSpec((Bn, 640), lambda i: (i, 0)),
            _full(ln2_g.shape), _full((1, 64)),
            _full(A1.shape), _full((1, 64)),
            _full(Ag.shape),
            _full(A2.shape), _full((1, 64)),
            _full(lnf_g.shape), _full((1, 64)),
        ],
        out_specs=pl.BlockSpec((Bn, 576), lambda i: (i, 0)),
        out_shape=jax.ShapeDtypeStruct((N, 576), f32),
    )(res, s2, ln2_g, ln2_b.reshape(1, -1), A1, bA1.reshape(1, -1), Ag, A2,
      bA2.reshape(1, -1), lnf_g, lnf_b.reshape(1, -1))

    return out.reshape(N, 9, C)
